# baseline (speedup 1.0000x reference)
import jax
import jax.numpy as jnp
from jax import lax
from jax.experimental import pallas as pl
from jax.experimental.pallas import tpu as pltpu
from jax.experimental.pallas import tpu_sc as plsc

VOCAB = 1000000
D = 32
ROWS = 16384
COLS = 200
B = ROWS * COLS
NC = 2
NS = 16
NW = NC * NS
ITPW = 4
C = ITPW * 128
NJ = COLS
SCALE = float(D) ** 0.5


def _body(xT_hbm, table_hbm, out_hbm, idx_v, rows_v, tbuf, isem, gsem, wsem):
    wid = lax.axis_index("s") * NC + lax.axis_index("c")
    colbase = wid * C
    i16 = lax.iota(jnp.int32, 16)

    def idx_copy(j, b):
        return pltpu.make_async_copy(
            xT_hbm.at[j, pl.ds(colbase, C)], idx_v.at[b], isem.at[b]
        )

    def gather_copy(b):
        return pltpu.make_async_copy(
            table_hbm.at[idx_v.at[b]], rows_v.at[b], gsem.at[b]
        )

    def write_copy(j, b):
        return pltpu.make_async_copy(
            tbuf.at[b], out_hbm.at[j, :, pl.ds(wid * ITPW, ITPW)], wsem.at[b]
        )

    idx_copy(0, 0).start()
    idx_copy(0, 0).wait()
    gather_copy(0).start()
    idx_copy(1, 1).start()

    @pl.loop(0, NJ)
    def _chunk(j):
        b = lax.rem(j, 2)
        nb = 1 - b

        @pl.when(j + 1 < NJ)
        def _():
            idx_copy(j + 1, nb).wait()
            gather_copy(nb).start()

        gather_copy(b).wait()

        @pl.when(j + 2 < NJ)
        def _():
            idx_copy(j + 2, b).start()

        @pl.when(j >= 2)
        def _():
            write_copy(j - 2, b).wait()

        rv = rows_v.at[b]
        tb = tbuf.at[b]

        @pl.loop(0, 4 * ITPW * 8)
        def _outer(o):
            dt = lax.shift_right_logical(o, 5)
            t = lax.bitwise_and(lax.shift_right_logical(o, 3), 3)
            r = lax.bitwise_and(o, 7)
            d = 8 * dt + r
            colv = jnp.full((16,), 0, jnp.int32) + d
            rowbase = t * 128
            for k in range(8):
                rowv = i16 + (rowbase + 16 * k)
                v = plsc.load_gather(rv, [rowv, colv])
                tb[dt, t, r, pl.ds(16 * k, 16)] = v * SCALE

        write_copy(j, b).start()

    write_copy(NJ - 2, 0).wait()
    write_copy(NJ - 1, 1).wait()


@jax.jit
def _embed(xT, table):
    mesh = plsc.VectorSubcoreMesh(
        core_axis_name="c", subcore_axis_name="s", num_cores=NC, num_subcores=NS
    )
    out5 = pl.kernel(
        _body,
        out_type=jax.ShapeDtypeStruct((NJ, 4, 128, 8, 128), jnp.float32),
        mesh=mesh,
        compiler_params=pltpu.CompilerParams(
            use_tc_tiling_on_sc=False, needs_layout_passes=False
        ),
        scratch_types=[
            pltpu.VMEM((2, C), jnp.int32),
            pltpu.VMEM((2, C, D), jnp.float32),
            pltpu.VMEM((2, 4, ITPW, 8, 128), jnp.float32),
            pltpu.SemaphoreType.DMA((2,)),
            pltpu.SemaphoreType.DMA((2,)),
            pltpu.SemaphoreType.DMA((2,)),
        ],
    )(xT.astype(jnp.int32), table)
    return out5.transpose(2, 4, 0, 1, 3).reshape(ROWS, COLS, D)


def kernel(x, table):
    return _embed(x.T, table)
